# baseline (speedup 1.0000x reference)
import jax
import jax.numpy as jnp
from jax import lax
from jax.experimental import pallas as pl
from jax.experimental.pallas import tpu as pltpu
from jax.experimental.pallas import tpu_sc as plsc

BATCH = 64
HW = 32
D = 256
NC = 2
NS = 16
WINDOW = 8


def _sc_body(row_hbm, col_hbm, out_hbm, chunk, sem):
    k = lax.axis_index("c") * NS + lax.axis_index("s")
    pltpu.sync_copy(col_hbm, chunk.at[:, pl.ds(0, D)])
    for i in range(HW):
        pltpu.sync_copy(row_hbm.at[pl.ds(k, 1)], chunk.at[pl.ds(i, 1), pl.ds(D, D)])
    pltpu.sync_copy(chunk, out_hbm.at[pl.ds(k * HW, HW)])
    src = out_hbm.at[pl.ds(k * HW, HW)]
    copies = [
        pltpu.make_async_copy(
            src, out_hbm.at[pl.ds(b * (HW * HW) + k * HW, HW)], sem
        )
        for b in range(1, BATCH)
    ]
    n = len(copies)
    for i in range(n):
        copies[i].start()
        if i >= WINDOW:
            copies[i - WINDOW].wait()
    for i in range(n - WINDOW, n):
        copies[i].wait()


def kernel(row_embeddings, column_embeddings):
    row = row_embeddings[:HW]
    col = column_embeddings[:HW]

    mesh = plsc.VectorSubcoreMesh(core_axis_name="c", subcore_axis_name="s")
    sc_kernel = pl.kernel(
        _sc_body,
        out_type=jax.ShapeDtypeStruct((BATCH * HW * HW, 2 * D), jnp.float32),
        mesh=mesh,
        scratch_types=[
            pltpu.VMEM((HW, 2 * D), jnp.float32),
            pltpu.SemaphoreType.DMA,
        ],
    )
    out = sc_kernel(row, col)
    return out.reshape(BATCH, HW * HW, 2 * D)


# speedup vs baseline: 25.9937x; 25.9937x over previous
import numpy as np

import jax
import jax.numpy as jnp
from jax.experimental import pallas as pl
from jax.sharding import Mesh, PartitionSpec as P

BATCH = 64
HW = 32
D = 256

BPB = 4


def _body(row_ref, col_ref, out_ref):
    col = col_ref[...]
    row = row_ref[...]
    out_ref[:, :, :, 0:D] = jax.lax.broadcast_in_dim(col, (BPB, HW, HW, D), (2, 3))
    out_ref[:, :, :, D : 2 * D] = jax.lax.broadcast_in_dim(
        row, (BPB, HW, HW, D), (1, 3)
    )


def _tc_call(row, col, nb):
    return pl.pallas_call(
        _body,
        grid=(nb // BPB,),
        in_specs=[
            pl.BlockSpec((HW, D), lambda b: (0, 0)),
            pl.BlockSpec((HW, D), lambda b: (0, 0)),
        ],
        out_specs=pl.BlockSpec((BPB, HW, HW, 2 * D), lambda b: (b, 0, 0, 0)),
        out_shape=jax.ShapeDtypeStruct((nb, HW, HW, 2 * D), jnp.float32),
    )(row, col)


def kernel(row_embeddings, column_embeddings):
    row = row_embeddings[:HW]
    col = column_embeddings[:HW]

    devs = jax.devices()
    nd = len(devs)
    while nd > 1 and BATCH % nd != 0:
        nd -= 1

    if nd > 1:
        mesh = Mesh(np.array(devs[:nd]), ("b",))
        out4 = jax.shard_map(
            lambda r, c: _tc_call(r, c, BATCH // nd),
            mesh=mesh,
            in_specs=(P(None, None), P(None, None)),
            out_specs=P("b", None, None, None),
            check_vma=False,
        )(row, col)
    else:
        out4 = _tc_call(row, col, BATCH)
    return out4.reshape(BATCH, HW * HW, 2 * D)
